# baseline (speedup 1.0000x reference)
import jax
import jax.numpy as jnp
from jax import lax
from jax.experimental import pallas as pl
from jax.experimental.pallas import tpu as pltpu
from jax.experimental.pallas import tpu_sc as plsc

_NUM_CORES = 2
_NUM_SUBCORES = 16
_NUM_WORKERS = _NUM_CORES * _NUM_SUBCORES

_B = 4
_T = 8192
_H = 1024
_ROWS_PER_WORKER = _T // _NUM_WORKERS
_SUBS = (112, 112, 32)
_BUF = max(_SUBS)
_G = 16
_NGROUPS = _B * _ROWS_PER_WORKER // _G


def _body(x_hbm, pe_hbm, out_hbm, buf, pe0, x_v, wsem):
    wid = lax.axis_index("s") * _NUM_CORES + lax.axis_index("c")
    base = wid * _ROWS_PER_WORKER

    pltpu.sync_copy(pe_hbm.at[pl.ds(0, 1)], pe0)
    pltpu.sync_copy(x_hbm.at[:, pl.ds(base, _ROWS_PER_WORKER)], x_v)

    writes = [
        pltpu.async_copy(pe_hbm.at[pl.ds(base, _ROWS_PER_WORKER)],
                         out_hbm.at[b, pl.ds(base, _ROWS_PER_WORKER)], wsem)
        for b in range(_B)
    ]

    acc = jnp.zeros((_G,), jnp.bool_)
    for b in range(_B):
        def scan(g, a, b=b):
            return a | (x_v[b, pl.ds(g * _G, _G)] == 0)

        acc = lax.fori_loop(0, _NGROUPS // _B, scan, acc)
    hit_any = jnp.any(acc)
    for h in writes:
        h.wait()

    @pl.when(hit_any)
    def _():
        for b in range(_B):
            def group(g, carry, b=b):
                xv = x_v[b, pl.ds(g * _G, _G)]

                @pl.when(jnp.any(xv == 0))
                def _():
                    for r in range(_G):
                        @pl.when(xv[r] == 0)
                        def _(b=b, r=r):
                            pltpu.sync_copy(
                                pe0,
                                out_hbm.at[b, pl.ds(base + g * _G + r, 1)])

                return carry

            lax.fori_loop(0, _NGROUPS // _B, group, 0)


@jax.jit
def kernel(x, pe):
    mesh = plsc.VectorSubcoreMesh(
        core_axis_name="c", subcore_axis_name="s",
        num_cores=_NUM_CORES, num_subcores=_NUM_SUBCORES)
    run = pl.kernel(
        _body,
        out_type=jax.ShapeDtypeStruct((_B, _T, _H), jnp.float32),
        mesh=mesh,
        compiler_params=pltpu.CompilerParams(needs_layout_passes=False),
        scratch_types=[
            pltpu.VMEM((_BUF, _H), jnp.float32),
            pltpu.VMEM((1, _H), jnp.float32),
            pltpu.VMEM((_B, _ROWS_PER_WORKER), jnp.int32),
            pltpu.SemaphoreType.DMA,
        ],
    )
    return run(x, pe)


# speedup vs baseline: 55.3180x; 55.3180x over previous
import jax
import jax.numpy as jnp
from jax import lax
from jax.experimental import pallas as pl
from jax.experimental.pallas import tpu as pltpu
from jax.experimental.pallas import tpu_sc as plsc

_NUM_CORES = 2
_NUM_SUBCORES = 16
_NUM_WORKERS = _NUM_CORES * _NUM_SUBCORES

_B = 4
_T = 8192
_H = 1024
_ROWS_PER_WORKER = _T // _NUM_WORKERS
_SUBS = (112, 112, 32)
_BUF = max(_SUBS)
_G = 16
_NGROUPS = _B * _ROWS_PER_WORKER // _G


def _body(x_hbm, pe_hbm, out_hbm, buf, pe0, x_v, wsem, xsem):
    wid = lax.axis_index("s") * _NUM_CORES + lax.axis_index("c")
    base = wid * _ROWS_PER_WORKER

    writes = []
    xread = None
    off = 0
    for sub in _SUBS:
        rb = base + off
        off += sub
        for h in writes:
            h.wait()
        pltpu.sync_copy(pe_hbm.at[pl.ds(rb, sub)], buf.at[pl.ds(0, sub)])
        writes = [
            pltpu.async_copy(buf.at[pl.ds(0, sub)],
                             out_hbm.at[b, pl.ds(rb, sub)], wsem)
            for b in range(_B)
        ]
        if xread is None:
            xread = pltpu.async_copy(
                x_hbm.at[:, pl.ds(base, _ROWS_PER_WORKER)], x_v, xsem)
    xread.wait()

    acc = jnp.zeros((_G,), jnp.bool_)
    for b in range(_B):
        def scan(g, a, b=b):
            return a | (x_v[b, pl.ds(g * _G, _G)] == 0)

        acc = lax.fori_loop(0, _NGROUPS // _B, scan, acc)
    hit_any = jnp.any(acc)
    for h in writes:
        h.wait()

    @pl.when(hit_any)
    def _():
        pltpu.sync_copy(pe_hbm.at[pl.ds(0, 1)], pe0)
        for b in range(_B):
            def group(g, carry, b=b):
                xv = x_v[b, pl.ds(g * _G, _G)]

                @pl.when(jnp.any(xv == 0))
                def _():
                    for r in range(_G):
                        @pl.when(xv[r] == 0)
                        def _(b=b, r=r):
                            pltpu.sync_copy(
                                pe0,
                                out_hbm.at[b, pl.ds(base + g * _G + r, 1)])

                return carry

            lax.fori_loop(0, _NGROUPS // _B, group, 0)


@jax.jit
def kernel(x, pe):
    mesh = plsc.VectorSubcoreMesh(
        core_axis_name="c", subcore_axis_name="s",
        num_cores=_NUM_CORES, num_subcores=_NUM_SUBCORES)
    run = pl.kernel(
        _body,
        out_type=jax.ShapeDtypeStruct((_B, _T, _H), jnp.float32),
        mesh=mesh,
        compiler_params=pltpu.CompilerParams(needs_layout_passes=False),
        scratch_types=[
            pltpu.VMEM((_BUF, _H), jnp.float32),
            pltpu.VMEM((1, _H), jnp.float32),
            pltpu.VMEM((_B, _ROWS_PER_WORKER), jnp.int32),
            pltpu.SemaphoreType.DMA,
            pltpu.SemaphoreType.DMA,
        ],
    )
    return run(x, pe)


# speedup vs baseline: 55.7302x; 1.0075x over previous
import jax
import jax.numpy as jnp
from jax import lax
from jax.experimental import pallas as pl
from jax.experimental.pallas import tpu as pltpu
from jax.experimental.pallas import tpu_sc as plsc

_NUM_CORES = 2
_NUM_SUBCORES = 16
_NUM_WORKERS = _NUM_CORES * _NUM_SUBCORES

_B = 4
_T = 8192
_H = 1024
_ROWS_PER_WORKER = _T // _NUM_WORKERS
_SUBS = (112, 112, 32)
_BUF = max(_SUBS)
_G = 16
_NGROUPS = _B * _ROWS_PER_WORKER // _G


def _body(x_hbm, pe_hbm, out_hbm, buf, pe0, x_v, wsem, xsem):
    wid = lax.axis_index("s") * _NUM_CORES + lax.axis_index("c")
    base = wid * _ROWS_PER_WORKER

    writes = []
    xread = None
    off = 0
    for sub in _SUBS:
        rb = base + off
        off += sub
        for h in writes:
            h.wait()
        pltpu.sync_copy(pe_hbm.at[pl.ds(rb, sub)], buf.at[pl.ds(0, sub)])
        writes = [
            pltpu.async_copy(
                buf.at[pl.ds(0, sub)],
                out_hbm.at[(wid + b) % _B, pl.ds(rb, sub)], wsem)
            for b in range(_B)
        ]
        if xread is None:
            xread = pltpu.async_copy(
                x_hbm.at[:, pl.ds(base, _ROWS_PER_WORKER)], x_v, xsem)
    xread.wait()

    acc = jnp.zeros((_G,), jnp.bool_)
    for b in range(_B):
        def scan(g, a, b=b):
            return a | (x_v[b, pl.ds(g * _G, _G)] == 0)

        acc = lax.fori_loop(0, _NGROUPS // _B, scan, acc)
    hit_any = jnp.any(acc)
    for h in writes:
        h.wait()

    @pl.when(hit_any)
    def _():
        pltpu.sync_copy(pe_hbm.at[pl.ds(0, 1)], pe0)
        for b in range(_B):
            def group(g, carry, b=b):
                xv = x_v[b, pl.ds(g * _G, _G)]

                @pl.when(jnp.any(xv == 0))
                def _():
                    for r in range(_G):
                        @pl.when(xv[r] == 0)
                        def _(b=b, r=r):
                            pltpu.sync_copy(
                                pe0,
                                out_hbm.at[b, pl.ds(base + g * _G + r, 1)])

                return carry

            lax.fori_loop(0, _NGROUPS // _B, group, 0)


@jax.jit
def kernel(x, pe):
    mesh = plsc.VectorSubcoreMesh(
        core_axis_name="c", subcore_axis_name="s",
        num_cores=_NUM_CORES, num_subcores=_NUM_SUBCORES)
    run = pl.kernel(
        _body,
        out_type=jax.ShapeDtypeStruct((_B, _T, _H), jnp.float32),
        mesh=mesh,
        compiler_params=pltpu.CompilerParams(needs_layout_passes=False),
        scratch_types=[
            pltpu.VMEM((_BUF, _H), jnp.float32),
            pltpu.VMEM((1, _H), jnp.float32),
            pltpu.VMEM((_B, _ROWS_PER_WORKER), jnp.int32),
            pltpu.SemaphoreType.DMA,
            pltpu.SemaphoreType.DMA,
        ],
    )
    return run(x, pe)
